# Initial kernel scaffold; baseline (speedup 1.0000x reference)
import jax
import jax.numpy as jnp
from jax.experimental import pallas as pl


def kernel(user_indices, item_indices, labels, head_indices, adj_entity, adj_relation, user_emb, item_emb, entity_emb, relation_emb, user_mlp_W, user_mlp_b, w_vv, w_ev, w_ve, w_ee, b_v, b_e, agg_W0, agg_b0, agg_W1, agg_b1):
    raise NotImplementedError("write your pallas kernel here")


# speedup vs baseline: 5.5482x; 5.5482x over previous
import functools

import jax
import jax.numpy as jnp
from jax import lax
from jax.experimental import pallas as pl
from jax.experimental.pallas import tpu as pltpu
from jax.experimental.pallas import tpu_sc as plsc

NC, NS = 2, 16
NW = NC * NS
L2W = 1e-06


def _sc_mesh():
    return plsc.VectorSubcoreMesh(core_axis_name="c", subcore_axis_name="s",
                                  num_cores=NC, num_subcores=NS)


_SC_PARAMS = pltpu.CompilerParams(use_tc_tiling_on_sc=False)


def _wid():
    return lax.axis_index("s") * NC + lax.axis_index("c")


def _sc_stage1(head, uidx, iidx, adj_e, adj_r, ent_emb, usr_emb, itm_emb):
    Bn = head.shape[0]
    nn = adj_e.shape[1]
    dim = ent_emb.shape[1]
    bpw = Bn // NW

    out_type = (
        jax.ShapeDtypeStruct((Bn, nn), jnp.int32),
        jax.ShapeDtypeStruct((Bn, nn), jnp.int32),
        jax.ShapeDtypeStruct((Bn, dim), jnp.float32),
        jax.ShapeDtypeStruct((Bn, dim), jnp.float32),
        jax.ShapeDtypeStruct((Bn, dim), jnp.float32),
    )

    @functools.partial(
        pl.kernel, out_type=out_type, mesh=_sc_mesh(),
        compiler_params=_SC_PARAMS,
        scratch_types=[
            pltpu.VMEM((bpw,), jnp.int32),
            pltpu.VMEM((bpw, nn), jnp.int32),
            pltpu.VMEM((bpw, dim), jnp.float32),
            pltpu.SemaphoreType.DMA,
        ],
    )
    def k(head_h, uidx_h, iidx_h, adj_e_h, adj_r_h, ent_h, usr_h, itm_h,
          e1_h, r1_h, ev0_h, uv_h, iv_h, idx_v, rows_i, rows_f, sem):
        base = _wid() * bpw
        sl = pl.ds(base, bpw)
        pltpu.sync_copy(head_h.at[sl], idx_v)
        pltpu.async_copy(adj_e_h.at[idx_v], rows_i, sem).wait()
        pltpu.sync_copy(rows_i, e1_h.at[sl])
        pltpu.async_copy(adj_r_h.at[idx_v], rows_i, sem).wait()
        pltpu.sync_copy(rows_i, r1_h.at[sl])
        pltpu.async_copy(ent_h.at[idx_v], rows_f, sem).wait()
        pltpu.sync_copy(rows_f, ev0_h.at[sl])
        pltpu.sync_copy(uidx_h.at[sl], idx_v)
        pltpu.async_copy(usr_h.at[idx_v], rows_f, sem).wait()
        pltpu.sync_copy(rows_f, uv_h.at[sl])
        pltpu.sync_copy(iidx_h.at[sl], idx_v)
        pltpu.async_copy(itm_h.at[idx_v], rows_f, sem).wait()
        pltpu.sync_copy(rows_f, iv_h.at[sl])

    return k(head, uidx, iidx, adj_e, adj_r, ent_emb, usr_emb, itm_emb)


def _sc_stage2(idx2d, adj_e, adj_r, ent_emb):
    nrows, W = idx2d.shape
    nn = adj_e.shape[1]
    dim = ent_emb.shape[1]
    rpw = nrows // NW
    nidx = nrows * W

    out_type = (
        jax.ShapeDtypeStruct((nidx, nn), jnp.int32),
        jax.ShapeDtypeStruct((nidx, nn), jnp.int32),
        jax.ShapeDtypeStruct((nidx, dim), jnp.float32),
    )

    @functools.partial(
        pl.kernel, out_type=out_type, mesh=_sc_mesh(),
        compiler_params=_SC_PARAMS,
        scratch_types=[
            pltpu.VMEM((W,), jnp.int32),
            pltpu.VMEM((W, nn), jnp.int32),
            pltpu.VMEM((W, dim), jnp.float32),
            pltpu.SemaphoreType.DMA,
        ],
    )
    def k(idx_h, adj_e_h, adj_r_h, ent_h, e2_h, r2_h, ev1_h,
          idx_v, rows_i, rows_f, sem):
        w0 = _wid() * rpw

        def body(j, _):
            row = w0 + j
            osl = pl.ds(row * W, W)
            pltpu.sync_copy(idx_h.at[row], idx_v)
            pltpu.async_copy(adj_e_h.at[idx_v], rows_i, sem).wait()
            pltpu.sync_copy(rows_i, e2_h.at[osl])
            pltpu.async_copy(adj_r_h.at[idx_v], rows_i, sem).wait()
            pltpu.sync_copy(rows_i, r2_h.at[osl])
            pltpu.async_copy(ent_h.at[idx_v], rows_f, sem).wait()
            pltpu.sync_copy(rows_f, ev1_h.at[osl])
            return 0

        lax.fori_loop(0, rpw, body, 0)

    return k(idx2d, adj_e, adj_r, ent_emb)


def _sc_stage3(idx2d, ent_emb):
    nrows, W = idx2d.shape
    dim = ent_emb.shape[1]
    rpw = nrows // NW

    out_type = jax.ShapeDtypeStruct((nrows * W, dim), jnp.float32)

    @functools.partial(
        pl.kernel, out_type=out_type, mesh=_sc_mesh(),
        compiler_params=_SC_PARAMS,
        scratch_types=[
            pltpu.VMEM((2, W), jnp.int32),
            pltpu.VMEM((2, W, dim), jnp.float32),
            pltpu.SemaphoreType.DMA,
            pltpu.SemaphoreType.DMA,
        ],
    )
    def k(idx_h, ent_h, out_h, idx_v, rows_v, sem0, sem1):
        w0 = _wid() * rpw
        sems = (sem0, sem1)

        def start(j, slot):
            pltpu.sync_copy(idx_h.at[w0 + j], idx_v.at[slot])
            return pltpu.async_copy(ent_h.at[idx_v.at[slot]], rows_v.at[slot],
                                    sems[slot])

        start(0, 0)
        start(1, 1)

        def body(j, _):
            slot = lax.rem(j, 2)

            def do(s):
                pltpu.make_async_copy(ent_h.at[idx_v.at[s]], rows_v.at[s],
                                      sems[s]).wait()
                pltpu.sync_copy(rows_v.at[s], out_h.at[pl.ds((w0 + j) * W, W)])

                @pl.when(j + 2 < rpw)
                def _():
                    start(j + 2, s)

            @pl.when(slot == 0)
            def _():
                do(0)

            @pl.when(slot == 1)
            def _():
                do(1)

            return 0

        lax.fori_loop(0, rpw, body, 0)

    return k(idx2d, ent_emb)


def _dense_kernel(uv_r, iv_r, ev0_r, ev1_r, ev2_r, r1_r, r2_r, lab_r,
                  rel_r, umw_r, umb_r, wvv_r, wev_r, wve_r, wee_r, bv_r, be_r,
                  w0_r, b0_r, w1_r, b1_r,
                  sig_r, bce_r, l2_r):
    i = pl.program_id(0)
    BB = uv_r.shape[0]
    nn = r1_r.shape[1]
    dim = uv_r.shape[1]
    nrel = rel_r.shape[0]

    u = uv_r[...]
    rel = rel_r[...]
    P = lax.dot_general(u, rel, (((1,), (1,)), ((), ())),
                        preferred_element_type=jnp.float32) * (1.0 / dim)

    r1 = r1_r[...]
    r2 = r2_r[...]
    S1 = jnp.zeros((BB, nn), jnp.float32)
    S2 = jnp.zeros((BB, nn * nn), jnp.float32)
    for r in range(nrel):
        pr = P[:, r:r + 1]
        S1 = jnp.where(r1 == r, pr, S1)
        S2 = jnp.where(r2 == r, pr, S2)

    def softmax(s):
        m = jnp.max(s, axis=-1, keepdims=True)
        e = jnp.exp(s - m)
        return e / jnp.sum(e, axis=-1, keepdims=True)

    w1 = softmax(S1)
    w2 = softmax(S2.reshape(BB, nn, nn))

    ev1 = ev1_r[...]
    ev2 = ev2_r[...].reshape(BB, nn, nn, dim)

    agg1 = jnp.zeros((BB, nn, dim), jnp.float32)
    for n in range(nn):
        agg1 = agg1 + w2[:, :, n:n + 1] * ev2[:, :, n, :]
    agg1 = agg1 * (1.0 / nn)
    W0 = w0_r[...]
    b0 = b0_r[...]
    h1 = jax.nn.relu(
        lax.dot_general((ev1 + agg1).reshape(BB * nn, dim), W0,
                        (((1,), (0,)), ((), ())),
                        preferred_element_type=jnp.float32)
        + b0).reshape(BB, nn, dim)

    agg0 = jnp.zeros((BB, dim), jnp.float32)
    for n in range(nn):
        agg0 = agg0 + w1[:, n:n + 1] * ev1[:, n, :]
    agg0 = agg0 * (1.0 / nn)
    h0 = jax.nn.relu(
        lax.dot_general(ev0_r[...] + agg0, W0, (((1,), (0,)), ((), ())),
                        preferred_element_type=jnp.float32) + b0)

    aggf = jnp.zeros((BB, dim), jnp.float32)
    for n in range(nn):
        aggf = aggf + w1[:, n:n + 1] * h1[:, n, :]
    aggf = aggf * (1.0 / nn)
    head = jnp.tanh(
        lax.dot_general(h0 + aggf, w1_r[...], (((1,), (0,)), ((), ())),
                        preferred_element_type=jnp.float32) + b1_r[...])

    uo = jax.nn.relu(
        lax.dot_general(u, umw_r[...], (((1,), (0,)), ((), ())),
                        preferred_element_type=jnp.float32) + umb_r[...])

    iv = iv_r[...]
    a1 = jnp.sum(head * wvv_r[...], axis=1, keepdims=True)
    a2 = jnp.sum(iv * wev_r[...], axis=1, keepdims=True)
    v_out = iv * a1 + head * a2 + bv_r[...]

    s = jnp.sum(uo * v_out, axis=1)
    sig_r[...] = (1.0 / (1.0 + jnp.exp(-s))).reshape(sig_r.shape)

    lab = lab_r[...].reshape(BB)
    bce = jnp.maximum(s, 0.0) - s * lab + jnp.log1p(jnp.exp(-jnp.abs(s)))

    @pl.when(i == 0)
    def _():
        psq = (jnp.sum(rel * rel)
               + jnp.sum(umw_r[...] ** 2) + jnp.sum(umb_r[...] ** 2)
               + jnp.sum(wvv_r[...] ** 2) + jnp.sum(wev_r[...] ** 2)
               + jnp.sum(wve_r[...] ** 2) + jnp.sum(wee_r[...] ** 2)
               + jnp.sum(bv_r[...] ** 2) + jnp.sum(be_r[...] ** 2)
               + jnp.sum(W0 * W0) + jnp.sum(b0 * b0)
               + jnp.sum(w1_r[...] ** 2) + jnp.sum(b1_r[...] ** 2))
        bce_r[...] = jnp.zeros((1, 1), jnp.float32)
        l2_r[...] = psq.reshape(1, 1)

    bce_r[...] += jnp.sum(bce).reshape(1, 1)
    l2_r[...] += (jnp.sum(uo * uo) + jnp.sum(v_out * v_out)).reshape(1, 1)


def _table_l2_kernel(a_r, b_r, c_r, acc_r):
    i = pl.program_id(0)

    @pl.when(i == 0)
    def _():
        acc_r[...] = jnp.zeros((1, 1), jnp.float32)

    a = a_r[...]
    b = b_r[...]
    c = c_r[...]
    acc_r[...] += (jnp.sum(a * a) + jnp.sum(b * b) + jnp.sum(c * c)).reshape(1, 1)


def kernel(user_indices, item_indices, labels, head_indices, adj_entity,
           adj_relation, user_emb, item_emb, entity_emb, relation_emb,
           user_mlp_W, user_mlp_b, w_vv, w_ev, w_ve, w_ee, b_v, b_e,
           agg_W0, agg_b0, agg_W1, agg_b1):
    Bn = user_indices.shape[0]
    dim = user_emb.shape[1]
    nn = adj_entity.shape[1]
    nrel = relation_emb.shape[0]

    e1, r1, ev0, uv, iv = _sc_stage1(
        head_indices, user_indices, item_indices,
        adj_entity, adj_relation, entity_emb, user_emb, item_emb)

    e1_idx = e1.reshape(Bn * nn // 128, 128)
    e2, r2, ev1 = _sc_stage2(e1_idx, adj_entity, adj_relation, entity_emb)

    e2_idx = e2.reshape(Bn * nn * nn // 128, 128)
    ev2 = _sc_stage3(e2_idx, entity_emb)

    BB = 32
    grid = Bn // BB
    ev1_3 = ev1.reshape(Bn, nn, dim)
    ev2_3 = ev2.reshape(Bn, nn * nn, dim)
    r2_2 = r2.reshape(Bn, nn * nn)
    lab_f = labels.astype(jnp.float32).reshape(grid, 1, BB)

    row = lambda x: x.reshape(1, dim)
    full = lambda shp: pl.BlockSpec(shp, lambda i: tuple(0 for _ in shp))

    sig, bce_sum, act_sq = pl.pallas_call(
        _dense_kernel,
        grid=(grid,),
        in_specs=[
            pl.BlockSpec((BB, dim), lambda i: (i, 0)),
            pl.BlockSpec((BB, dim), lambda i: (i, 0)),
            pl.BlockSpec((BB, dim), lambda i: (i, 0)),
            pl.BlockSpec((BB, nn, dim), lambda i: (i, 0, 0)),
            pl.BlockSpec((BB, nn * nn, dim), lambda i: (i, 0, 0)),
            pl.BlockSpec((BB, nn), lambda i: (i, 0)),
            pl.BlockSpec((BB, nn * nn), lambda i: (i, 0)),
            pl.BlockSpec((1, 1, BB), lambda i: (i, 0, 0)),
            full((nrel, dim)),
            full((dim, dim)),
            full((1, dim)),
            full((1, dim)),
            full((1, dim)),
            full((1, dim)),
            full((1, dim)),
            full((1, dim)),
            full((1, dim)),
            full((dim, dim)),
            full((1, dim)),
            full((dim, dim)),
            full((1, dim)),
        ],
        out_specs=[
            pl.BlockSpec((1, 1, BB), lambda i: (i, 0, 0)),
            pl.BlockSpec((1, 1), lambda i: (0, 0)),
            pl.BlockSpec((1, 1), lambda i: (0, 0)),
        ],
        out_shape=[
            jax.ShapeDtypeStruct((grid, 1, BB), jnp.float32),
            jax.ShapeDtypeStruct((1, 1), jnp.float32),
            jax.ShapeDtypeStruct((1, 1), jnp.float32),
        ],
    )(uv, iv, ev0, ev1_3, ev2_3, r1, r2_2, lab_f,
      relation_emb, user_mlp_W, row(user_mlp_b),
      row(w_vv.reshape(dim)), row(w_ev.reshape(dim)),
      row(w_ve.reshape(dim)), row(w_ee.reshape(dim)),
      row(b_v), row(b_e),
      agg_W0, row(agg_b0), agg_W1, row(agg_b1))

    RB = 2000
    tgrid = user_emb.shape[0] // RB
    tab_sq = pl.pallas_call(
        _table_l2_kernel,
        grid=(tgrid,),
        in_specs=[
            pl.BlockSpec((RB, dim), lambda i: (i, 0)),
            pl.BlockSpec((RB, dim), lambda i: (i, 0)),
            pl.BlockSpec((RB, dim), lambda i: (i, 0)),
        ],
        out_specs=pl.BlockSpec((1, 1), lambda i: (0, 0)),
        out_shape=jax.ShapeDtypeStruct((1, 1), jnp.float32),
    )(user_emb, item_emb, entity_emb)

    scores_normalized = sig.reshape(Bn)
    total_sq = act_sq[0, 0] + tab_sq[0, 0]
    loss = bce_sum[0, 0] / Bn + (0.5 * L2W) * total_sq
    return (scores_normalized, loss)


# speedup vs baseline: 6.2799x; 1.1319x over previous
import functools

import jax
import jax.numpy as jnp
from jax import lax
from jax.experimental import pallas as pl
from jax.experimental.pallas import tpu as pltpu
from jax.experimental.pallas import tpu_sc as plsc

NC, NS = 2, 16
NW = NC * NS
L2W = 1e-06


def _sc_mesh():
    return plsc.VectorSubcoreMesh(core_axis_name="c", subcore_axis_name="s",
                                  num_cores=NC, num_subcores=NS)


_SC_PARAMS = pltpu.CompilerParams(use_tc_tiling_on_sc=False)


def _wid():
    return lax.axis_index("s") * NC + lax.axis_index("c")


def _sc_stage1(head, adj_e, adj_r, ent_emb):
    Bn = head.shape[0]
    nn = adj_e.shape[1]
    dim = ent_emb.shape[1]
    bpw = Bn // NW

    out_type = (
        jax.ShapeDtypeStruct((Bn, nn), jnp.int32),
        jax.ShapeDtypeStruct((Bn, nn), jnp.int32),
        jax.ShapeDtypeStruct((Bn, dim), jnp.float32),
    )

    @functools.partial(
        pl.kernel, out_type=out_type, mesh=_sc_mesh(),
        compiler_params=_SC_PARAMS,
        scratch_types=[
            pltpu.VMEM((bpw,), jnp.int32),
            pltpu.VMEM((bpw, nn), jnp.int32),
            pltpu.VMEM((bpw, dim), jnp.float32),
            pltpu.SemaphoreType.DMA,
        ],
    )
    def k(head_h, adj_e_h, adj_r_h, ent_h,
          e1_h, r1_h, ev0_h, idx_v, rows_i, rows_f, sem):
        base = _wid() * bpw
        sl = pl.ds(base, bpw)
        pltpu.sync_copy(head_h.at[sl], idx_v)
        pltpu.async_copy(adj_e_h.at[idx_v], rows_i, sem).wait()
        pltpu.sync_copy(rows_i, e1_h.at[sl])
        pltpu.async_copy(adj_r_h.at[idx_v], rows_i, sem).wait()
        pltpu.sync_copy(rows_i, r1_h.at[sl])
        pltpu.async_copy(ent_h.at[idx_v], rows_f, sem).wait()
        pltpu.sync_copy(rows_f, ev0_h.at[sl])

    return k(head, adj_e, adj_r, ent_emb)


def _sc_stage_ui(uidx, iidx, usr_emb, itm_emb):
    Bn = uidx.shape[0]
    dim = usr_emb.shape[1]
    bpw = Bn // NW

    out_type = (
        jax.ShapeDtypeStruct((Bn, dim), jnp.float32),
        jax.ShapeDtypeStruct((Bn, dim), jnp.float32),
    )

    @functools.partial(
        pl.kernel, out_type=out_type, mesh=_sc_mesh(),
        compiler_params=_SC_PARAMS,
        scratch_types=[
            pltpu.VMEM((bpw,), jnp.int32),
            pltpu.VMEM((bpw, dim), jnp.float32),
            pltpu.SemaphoreType.DMA,
        ],
    )
    def k(uidx_h, iidx_h, usr_h, itm_h, uv_h, iv_h, idx_v, rows_f, sem):
        base = _wid() * bpw
        sl = pl.ds(base, bpw)
        pltpu.sync_copy(uidx_h.at[sl], idx_v)
        pltpu.async_copy(usr_h.at[idx_v], rows_f, sem).wait()
        pltpu.sync_copy(rows_f, uv_h.at[sl])
        pltpu.sync_copy(iidx_h.at[sl], idx_v)
        pltpu.async_copy(itm_h.at[idx_v], rows_f, sem).wait()
        pltpu.sync_copy(rows_f, iv_h.at[sl])

    return k(uidx, iidx, usr_emb, itm_emb)


def _sc_stage2(idx2d, adj_e, adj_r, ent_emb):
    nrows, W = idx2d.shape
    nn = adj_e.shape[1]
    dim = ent_emb.shape[1]
    rpw = nrows // NW
    nidx = nrows * W

    out_type = (
        jax.ShapeDtypeStruct((nidx, nn), jnp.int32),
        jax.ShapeDtypeStruct((nidx, nn), jnp.int32),
        jax.ShapeDtypeStruct((nidx, dim), jnp.float32),
    )

    @functools.partial(
        pl.kernel, out_type=out_type, mesh=_sc_mesh(),
        compiler_params=_SC_PARAMS,
        scratch_types=[
            pltpu.VMEM((W,), jnp.int32),
            pltpu.VMEM((W, nn), jnp.int32),
            pltpu.VMEM((W, dim), jnp.float32),
            pltpu.SemaphoreType.DMA,
        ],
    )
    def k(idx_h, adj_e_h, adj_r_h, ent_h, e2_h, r2_h, ev1_h,
          idx_v, rows_i, rows_f, sem):
        w0 = _wid() * rpw

        def body(j, _):
            row = w0 + j
            osl = pl.ds(row * W, W)
            pltpu.sync_copy(idx_h.at[row], idx_v)
            pltpu.async_copy(adj_e_h.at[idx_v], rows_i, sem).wait()
            pltpu.sync_copy(rows_i, e2_h.at[osl])
            pltpu.async_copy(adj_r_h.at[idx_v], rows_i, sem).wait()
            pltpu.sync_copy(rows_i, r2_h.at[osl])
            pltpu.async_copy(ent_h.at[idx_v], rows_f, sem).wait()
            pltpu.sync_copy(rows_f, ev1_h.at[osl])
            return 0

        lax.fori_loop(0, rpw, body, 0)

    return k(idx2d, adj_e, adj_r, ent_emb)


def _sc_stage3(idx2d, ent_emb):
    nrows, W = idx2d.shape
    dim = ent_emb.shape[1]
    rpw = nrows // NW

    out_type = jax.ShapeDtypeStruct((nrows * W, dim), jnp.float32)

    @functools.partial(
        pl.kernel, out_type=out_type, mesh=_sc_mesh(),
        compiler_params=_SC_PARAMS,
        scratch_types=[
            pltpu.VMEM((2, W), jnp.int32),
            pltpu.VMEM((2, W, dim), jnp.float32),
            pltpu.SemaphoreType.DMA,
            pltpu.SemaphoreType.DMA,
        ],
    )
    def k(idx_h, ent_h, out_h, idx_v, rows_v, sem0, sem1):
        w0 = _wid() * rpw
        sems = (sem0, sem1)

        def start(j, slot):
            pltpu.sync_copy(idx_h.at[w0 + j], idx_v.at[slot])
            return pltpu.async_copy(ent_h.at[idx_v.at[slot]], rows_v.at[slot],
                                    sems[slot])

        start(0, 0)
        start(1, 1)

        def body(j, _):
            slot = lax.rem(j, 2)

            def do(s):
                pltpu.make_async_copy(ent_h.at[idx_v.at[s]], rows_v.at[s],
                                      sems[s]).wait()
                pltpu.sync_copy(rows_v.at[s], out_h.at[pl.ds((w0 + j) * W, W)])

                @pl.when(j + 2 < rpw)
                def _():
                    start(j + 2, s)

            @pl.when(slot == 0)
            def _():
                do(0)

            @pl.when(slot == 1)
            def _():
                do(1)

            return 0

        lax.fori_loop(0, rpw, body, 0)

    return k(idx2d, ent_emb)


def _dense_kernel(uv_r, iv_r, ev0_r, ev1_r, ev2_r, r1_r, r2_r, lab_r,
                  rel_r, umw_r, umb_r, wvv_r, wev_r, wve_r, wee_r, bv_r, be_r,
                  w0_r, b0_r, w1_r, b1_r,
                  sig_r, bce_r, l2_r):
    i = pl.program_id(0)
    BB = uv_r.shape[0]
    dim = uv_r.shape[1]
    nn = 16
    nrel = rel_r.shape[0]
    GG = BB * nn

    u = uv_r[...]
    rel = rel_r[...]
    P = lax.dot_general(u, rel, (((1,), (1,)), ((), ())),
                        preferred_element_type=jnp.float32) * (1.0 / dim)

    r1i = r1_r[...]
    r2i = r2_r[...]

    rows = lax.broadcasted_iota(jnp.int32, (GG, BB), 0) // nn
    cols = lax.broadcasted_iota(jnp.int32, (GG, BB), 1)
    RepM = (rows == cols).astype(jnp.float32)
    P_exp = lax.dot_general(RepM, P, (((1,), (0,)), ((), ())),
                            preferred_element_type=jnp.float32)

    S1 = jnp.zeros((BB, nn), jnp.float32)
    S2 = jnp.zeros((GG, nn), jnp.float32)
    for r in range(nrel):
        S1 = jnp.where(r1i == r, P[:, r:r + 1], S1)
        S2 = jnp.where(r2i == r, P_exp[:, r:r + 1], S2)

    def softmax(s):
        m = jnp.max(s, axis=-1, keepdims=True)
        e = jnp.exp(s - m)
        return e / jnp.sum(e, axis=-1, keepdims=True)

    w1 = softmax(S1)
    w2 = softmax(S2)

    en = lax.broadcasted_iota(jnp.int32, (nn, 8, 2 * dim), 0)
    es = lax.broadcasted_iota(jnp.int32, (nn, 8, 2 * dim), 1)
    el = lax.broadcasted_iota(jnp.int32, (nn, 8, 2 * dim), 2)
    E3 = (en == (es * 2 + el // dim)).astype(jnp.float32)

    W23 = lax.dot_general(w2, E3, (((1,), (0,)), ((), ())),
                          preferred_element_type=jnp.float32)
    ev2 = ev2_r[...].reshape(GG, 8, 2 * dim)
    aggp = jnp.sum(ev2 * W23, axis=1)
    agg1 = (aggp[:, :dim] + aggp[:, dim:]) * (1.0 / nn)
    X2 = jnp.concatenate([agg1, agg1], axis=1)
    mq = lax.broadcasted_iota(jnp.int32, (GG // 2, GG), 0)
    mg = lax.broadcasted_iota(jnp.int32, (GG // 2, GG), 1)
    Meven = (mg == 2 * mq).astype(jnp.float32)
    Modd = (mg == 2 * mq + 1).astype(jnp.float32)
    lmask = (lax.broadcasted_iota(jnp.int32, (1, 2 * dim), 1)
             < dim).astype(jnp.float32)
    agg1p = (lax.dot_general(Meven, X2, (((1,), (0,)), ((), ())),
                             preferred_element_type=jnp.float32) * lmask
             + lax.dot_general(Modd, X2, (((1,), (0,)), ((), ())),
                               preferred_element_type=jnp.float32)
             * (1.0 - lmask))

    W0 = w0_r[...]
    z64 = jnp.zeros((dim, dim), jnp.float32)
    W0bd = jnp.concatenate(
        [jnp.concatenate([W0, z64], axis=1),
         jnp.concatenate([z64, W0], axis=1)], axis=0)
    b0 = b0_r[...]
    b0p = jnp.concatenate([b0, b0], axis=1)

    ev1 = ev1_r[...]
    h1p = jax.nn.relu(
        lax.dot_general(ev1 + agg1p, W0bd, (((1,), (0,)), ((), ())),
                        preferred_element_type=jnp.float32) + b0p)

    W13 = lax.dot_general(w1, E3, (((1,), (0,)), ((), ())),
                          preferred_element_type=jnp.float32)

    def hop0_agg(xp):
        s = jnp.sum(xp.reshape(BB, 8, 2 * dim) * W13, axis=1)
        return (s[:, :dim] + s[:, dim:]) * (1.0 / nn)

    agg0 = hop0_agg(ev1)
    h0 = jax.nn.relu(
        lax.dot_general(ev0_r[...] + agg0, W0, (((1,), (0,)), ((), ())),
                        preferred_element_type=jnp.float32) + b0)

    aggf = hop0_agg(h1p)
    head = jnp.tanh(
        lax.dot_general(h0 + aggf, w1_r[...], (((1,), (0,)), ((), ())),
                        preferred_element_type=jnp.float32) + b1_r[...])

    uo = jax.nn.relu(
        lax.dot_general(u, umw_r[...], (((1,), (0,)), ((), ())),
                        preferred_element_type=jnp.float32) + umb_r[...])

    iv = iv_r[...]
    a1 = jnp.sum(head * wvv_r[...], axis=1, keepdims=True)
    a2 = jnp.sum(iv * wev_r[...], axis=1, keepdims=True)
    v_out = iv * a1 + head * a2 + bv_r[...]

    s = jnp.sum(uo * v_out, axis=1)
    sig_r[...] = (1.0 / (1.0 + jnp.exp(-s))).reshape(sig_r.shape)

    lab = lab_r[...].reshape(BB)
    bce = jnp.maximum(s, 0.0) - s * lab + jnp.log1p(jnp.exp(-jnp.abs(s)))

    @pl.when(i == 0)
    def _():
        psq = (jnp.sum(rel * rel)
               + jnp.sum(umw_r[...] ** 2) + jnp.sum(umb_r[...] ** 2)
               + jnp.sum(wvv_r[...] ** 2) + jnp.sum(wev_r[...] ** 2)
               + jnp.sum(wve_r[...] ** 2) + jnp.sum(wee_r[...] ** 2)
               + jnp.sum(bv_r[...] ** 2) + jnp.sum(be_r[...] ** 2)
               + jnp.sum(W0 * W0) + jnp.sum(b0 * b0)
               + jnp.sum(w1_r[...] ** 2) + jnp.sum(b1_r[...] ** 2))
        bce_r[...] = jnp.zeros((1, 1), jnp.float32)
        l2_r[...] = psq.reshape(1, 1)

    bce_r[...] += jnp.sum(bce).reshape(1, 1)
    l2_r[...] += (jnp.sum(uo * uo) + jnp.sum(v_out * v_out)).reshape(1, 1)


def _table_l2_kernel(a_r, b_r, c_r, acc_r):
    i = pl.program_id(0)

    @pl.when(i == 0)
    def _():
        acc_r[...] = jnp.zeros((1, 1), jnp.float32)

    a = a_r[...]
    b = b_r[...]
    c = c_r[...]
    acc_r[...] += (jnp.sum(a * a) + jnp.sum(b * b) + jnp.sum(c * c)).reshape(1, 1)


def kernel(user_indices, item_indices, labels, head_indices, adj_entity,
           adj_relation, user_emb, item_emb, entity_emb, relation_emb,
           user_mlp_W, user_mlp_b, w_vv, w_ev, w_ve, w_ee, b_v, b_e,
           agg_W0, agg_b0, agg_W1, agg_b1):
    Bn = user_indices.shape[0]
    dim = user_emb.shape[1]
    nn = adj_entity.shape[1]
    nrel = relation_emb.shape[0]

    e1, r1, ev0 = _sc_stage1(head_indices, adj_entity, adj_relation,
                             entity_emb)

    e1_idx = e1.reshape(Bn * nn // 128, 128)
    e2, r2, ev1 = _sc_stage2(e1_idx, adj_entity, adj_relation, entity_emb)

    e2_idx = e2.reshape(Bn * nn * nn // 128, 128)
    ev2 = _sc_stage3(e2_idx, entity_emb)

    uv, iv = _sc_stage_ui(user_indices, item_indices, user_emb, item_emb)

    BB = 32
    grid = Bn // BB
    ev1_p = ev1.reshape(Bn * nn * dim // 128, 128)
    ev2_p = ev2.reshape(Bn * nn * nn * dim // 128, 128)
    r2_g = r2.reshape(Bn * nn, nn)
    lab_f = labels.astype(jnp.float32).reshape(grid, 1, BB)

    row = lambda x: x.reshape(1, dim)
    full = lambda shp: pl.BlockSpec(shp, lambda i: tuple(0 for _ in shp))

    sig, bce_sum, act_sq = pl.pallas_call(
        _dense_kernel,
        grid=(grid,),
        in_specs=[
            pl.BlockSpec((BB, dim), lambda i: (i, 0)),
            pl.BlockSpec((BB, dim), lambda i: (i, 0)),
            pl.BlockSpec((BB, dim), lambda i: (i, 0)),
            pl.BlockSpec((BB * nn * dim // 128, 128), lambda i: (i, 0)),
            pl.BlockSpec((BB * nn * nn * dim // 128, 128), lambda i: (i, 0)),
            pl.BlockSpec((BB, nn), lambda i: (i, 0)),
            pl.BlockSpec((BB * nn, nn), lambda i: (i, 0)),
            pl.BlockSpec((1, 1, BB), lambda i: (i, 0, 0)),
            full((nrel, dim)),
            full((dim, dim)),
            full((1, dim)),
            full((1, dim)),
            full((1, dim)),
            full((1, dim)),
            full((1, dim)),
            full((1, dim)),
            full((1, dim)),
            full((dim, dim)),
            full((1, dim)),
            full((dim, dim)),
            full((1, dim)),
        ],
        out_specs=[
            pl.BlockSpec((1, 1, BB), lambda i: (i, 0, 0)),
            pl.BlockSpec((1, 1), lambda i: (0, 0)),
            pl.BlockSpec((1, 1), lambda i: (0, 0)),
        ],
        out_shape=[
            jax.ShapeDtypeStruct((grid, 1, BB), jnp.float32),
            jax.ShapeDtypeStruct((1, 1), jnp.float32),
            jax.ShapeDtypeStruct((1, 1), jnp.float32),
        ],
    )(uv, iv, ev0, ev1_p, ev2_p, r1, r2_g, lab_f,
      relation_emb, user_mlp_W, row(user_mlp_b),
      row(w_vv.reshape(dim)), row(w_ev.reshape(dim)),
      row(w_ve.reshape(dim)), row(w_ee.reshape(dim)),
      row(b_v), row(b_e),
      agg_W0, row(agg_b0), agg_W1, row(agg_b1))

    RB = 2000
    tgrid = user_emb.shape[0] // RB
    tab_sq = pl.pallas_call(
        _table_l2_kernel,
        grid=(tgrid,),
        in_specs=[
            pl.BlockSpec((RB, dim), lambda i: (i, 0)),
            pl.BlockSpec((RB, dim), lambda i: (i, 0)),
            pl.BlockSpec((RB, dim), lambda i: (i, 0)),
        ],
        out_specs=pl.BlockSpec((1, 1), lambda i: (0, 0)),
        out_shape=jax.ShapeDtypeStruct((1, 1), jnp.float32),
    )(user_emb, item_emb, entity_emb)

    scores_normalized = sig.reshape(Bn)
    total_sq = act_sq[0, 0] + tab_sq[0, 0]
    loss = bce_sum[0, 0] / Bn + (0.5 * L2W) * total_sq
    return (scores_normalized, loss)


# speedup vs baseline: 6.7083x; 1.0682x over previous
import functools

import jax
import jax.numpy as jnp
from jax import lax
from jax.experimental import pallas as pl
from jax.experimental.pallas import tpu as pltpu
from jax.experimental.pallas import tpu_sc as plsc

NC, NS = 2, 16
NW = NC * NS
L2W = 1e-06


def _sc_mesh():
    return plsc.VectorSubcoreMesh(core_axis_name="c", subcore_axis_name="s",
                                  num_cores=NC, num_subcores=NS)


_SC_PARAMS = pltpu.CompilerParams(use_tc_tiling_on_sc=False)
_SC_PARAMS_V = pltpu.CompilerParams(use_tc_tiling_on_sc=False,
                                    needs_layout_passes=False)


def _wid():
    return lax.axis_index("s") * NC + lax.axis_index("c")


def _sc_stage1(head, adj_e, adj_r, ent_emb):
    Bn = head.shape[0]
    nn = adj_e.shape[1]
    dim = ent_emb.shape[1]
    bpw = Bn // NW

    out_type = (
        jax.ShapeDtypeStruct((Bn, nn), jnp.int32),
        jax.ShapeDtypeStruct((Bn, nn), jnp.int32),
        jax.ShapeDtypeStruct((Bn, dim), jnp.float32),
    )

    @functools.partial(
        pl.kernel, out_type=out_type, mesh=_sc_mesh(),
        compiler_params=_SC_PARAMS,
        scratch_types=[
            pltpu.VMEM((bpw,), jnp.int32),
            pltpu.VMEM((bpw, nn), jnp.int32),
            pltpu.VMEM((bpw, dim), jnp.float32),
            pltpu.SemaphoreType.DMA,
        ],
    )
    def k(head_h, adj_e_h, adj_r_h, ent_h,
          e1_h, r1_h, ev0_h, idx_v, rows_i, rows_f, sem):
        base = _wid() * bpw
        sl = pl.ds(base, bpw)
        pltpu.sync_copy(head_h.at[sl], idx_v)
        pltpu.async_copy(adj_e_h.at[idx_v], rows_i, sem).wait()
        pltpu.sync_copy(rows_i, e1_h.at[sl])
        pltpu.async_copy(adj_r_h.at[idx_v], rows_i, sem).wait()
        pltpu.sync_copy(rows_i, r1_h.at[sl])
        pltpu.async_copy(ent_h.at[idx_v], rows_f, sem).wait()
        pltpu.sync_copy(rows_f, ev0_h.at[sl])

    return k(head, adj_e, adj_r, ent_emb)


def _sc_stage_ui(uidx, iidx, usr_emb, itm_emb):
    Bn = uidx.shape[0]
    dim = usr_emb.shape[1]
    bpw = Bn // NW

    out_type = (
        jax.ShapeDtypeStruct((Bn, dim), jnp.float32),
        jax.ShapeDtypeStruct((Bn, dim), jnp.float32),
    )

    @functools.partial(
        pl.kernel, out_type=out_type, mesh=_sc_mesh(),
        compiler_params=_SC_PARAMS,
        scratch_types=[
            pltpu.VMEM((bpw,), jnp.int32),
            pltpu.VMEM((bpw, dim), jnp.float32),
            pltpu.SemaphoreType.DMA,
        ],
    )
    def k(uidx_h, iidx_h, usr_h, itm_h, uv_h, iv_h, idx_v, rows_f, sem):
        base = _wid() * bpw
        sl = pl.ds(base, bpw)
        pltpu.sync_copy(uidx_h.at[sl], idx_v)
        pltpu.async_copy(usr_h.at[idx_v], rows_f, sem).wait()
        pltpu.sync_copy(rows_f, uv_h.at[sl])
        pltpu.sync_copy(iidx_h.at[sl], idx_v)
        pltpu.async_copy(itm_h.at[idx_v], rows_f, sem).wait()
        pltpu.sync_copy(rows_f, iv_h.at[sl])

    return k(uidx, iidx, usr_emb, itm_emb)


def _sc_stage2(idx2d, adj_e, adj_r, ent_emb):
    nrows, W = idx2d.shape
    nn = adj_e.shape[1]
    dim = ent_emb.shape[1]
    rpw = nrows // NW
    nidx = nrows * W

    out_type = (
        jax.ShapeDtypeStruct((nidx, nn), jnp.int32),
        jax.ShapeDtypeStruct((nidx, nn), jnp.int32),
        jax.ShapeDtypeStruct((nidx, dim), jnp.float32),
    )

    @functools.partial(
        pl.kernel, out_type=out_type, mesh=_sc_mesh(),
        compiler_params=_SC_PARAMS,
        scratch_types=[
            pltpu.VMEM((W,), jnp.int32),
            pltpu.VMEM((W, nn), jnp.int32),
            pltpu.VMEM((W, dim), jnp.float32),
            pltpu.SemaphoreType.DMA,
        ],
    )
    def k(idx_h, adj_e_h, adj_r_h, ent_h, e2_h, r2_h, ev1_h,
          idx_v, rows_i, rows_f, sem):
        w0 = _wid() * rpw

        def body(j, _):
            row = w0 + j
            osl = pl.ds(row * W, W)
            pltpu.sync_copy(idx_h.at[row], idx_v)
            pltpu.async_copy(adj_e_h.at[idx_v], rows_i, sem).wait()
            pltpu.sync_copy(rows_i, e2_h.at[osl])
            pltpu.async_copy(adj_r_h.at[idx_v], rows_i, sem).wait()
            pltpu.sync_copy(rows_i, r2_h.at[osl])
            pltpu.async_copy(ent_h.at[idx_v], rows_f, sem).wait()
            pltpu.sync_copy(rows_f, ev1_h.at[osl])
            return 0

        lax.fori_loop(0, rpw, body, 0)

    return k(idx2d, adj_e, adj_r, ent_emb)


def _sc_stage3(idx2d, r2, p_mat, ent_emb):
    nrows, W = idx2d.shape
    ng, nn = r2.shape
    dim = ent_emb.shape[1]
    nrel = 32
    rpw = nrows // NW
    gpw = ng // NW
    gpc = W // nn
    bpw = p_mat.shape[0] // nrel // NW
    OB = 16

    out_type = jax.ShapeDtypeStruct((ng, dim), jnp.float32)

    @functools.partial(
        pl.kernel, out_type=out_type, mesh=_sc_mesh(),
        compiler_params=_SC_PARAMS_V,
        scratch_types=[
            pltpu.VMEM((gpw, nn), jnp.int32),
            pltpu.VMEM((bpw * nrel,), jnp.float32),
            pltpu.VMEM((2, W), jnp.int32),
            pltpu.VMEM((2, W, dim), jnp.float32),
            pltpu.VMEM((OB * gpc, dim), jnp.float32),
            pltpu.VMEM((nn,), jnp.float32),
            pltpu.SemaphoreType.DMA,
            pltpu.SemaphoreType.DMA,
            pltpu.SemaphoreType.DMA,
        ],
    )
    def k(idx_h, r2_h, p_h, ent_h, out_h,
          r2_v, p_v, idx_v, rows_v, out_v, w_v, sem0, sem1, semo):
        wid = _wid()
        w0 = wid * rpw
        pltpu.sync_copy(r2_h.at[pl.ds(wid * gpw, gpw)], r2_v)
        pltpu.sync_copy(p_h.at[pl.ds(wid * bpw * nrel, bpw * nrel)], p_v)
        sems = (sem0, sem1)

        def start(j, slot):
            pltpu.sync_copy(idx_h.at[w0 + j], idx_v.at[slot])
            return pltpu.async_copy(ent_h.at[idx_v.at[slot]], rows_v.at[slot],
                                    sems[slot])

        start(0, 0)
        start(1, 1)

        def chunk_body(j, _):
            slot = lax.rem(j, 2)

            def do(s):
                pltpu.make_async_copy(ent_h.at[idx_v.at[s]], rows_v.at[s],
                                      sems[s]).wait()
                obase = lax.rem(j, OB) * gpc
                for g in range(gpc):
                    gl = j * gpc + g
                    bl = gl // nn
                    rvec = r2_v[gl, :]
                    svec = plsc.load_gather(
                        p_v, [bl * nrel + rvec])
                    m = jnp.max(svec, axis=0)
                    e = jnp.exp(svec - m)
                    tot = jnp.sum(e, axis=0) * float(nn)
                    w_v[...] = e / tot
                    for db in range(dim // 16):
                        sl = pl.ds(db * 16, 16)
                        acc = jnp.zeros((16,), jnp.float32)
                        for n in range(nn):
                            wn = plsc.load_gather(
                                w_v, [jnp.full((16,), n, jnp.int32)])
                            acc = acc + wn * rows_v[s, g * nn + n, sl]
                        out_v[obase + g, sl] = acc

                @pl.when(j + 2 < rpw)
                def _():
                    start(j + 2, s)

                @pl.when(lax.rem(j, OB) == OB - 1)
                def _():
                    pltpu.async_copy(
                        out_v,
                        out_h.at[pl.ds(wid * gpw + (j - (OB - 1)) * gpc,
                                       OB * gpc)],
                        semo).wait()

            @pl.when(slot == 0)
            def _():
                do(0)

            @pl.when(slot == 1)
            def _():
                do(1)

            return 0

        lax.fori_loop(0, rpw, chunk_body, 0)

    return k(idx2d, r2, p_mat, ent_emb)


def _p_kernel(uv_r, rel_r, p_r):
    dim = uv_r.shape[1]
    p_r[...] = lax.dot_general(uv_r[...], rel_r[...], (((1,), (1,)), ((), ())),
                               preferred_element_type=jnp.float32) * (1.0 / dim)


def _dense_kernel(uv_r, iv_r, ev0_r, ev1_r, agg1_r, r1_r, p_r, lab_r,
                  rel_r, umw_r, umb_r, wvv_r, wev_r, wve_r, wee_r, bv_r, be_r,
                  w0_r, b0_r, w1_r, b1_r,
                  sig_r, bce_r, l2_r):
    i = pl.program_id(0)
    BB = uv_r.shape[0]
    dim = uv_r.shape[1]
    nn = 16
    nrel = rel_r.shape[0]

    u = uv_r[...]
    rel = rel_r[...]
    P = p_r[...]

    r1i = r1_r[...]
    S1 = jnp.zeros((BB, nn), jnp.float32)
    for r in range(nrel):
        S1 = jnp.where(r1i == r, P[:, r:r + 1], S1)
    m = jnp.max(S1, axis=-1, keepdims=True)
    e = jnp.exp(S1 - m)
    w1 = e / jnp.sum(e, axis=-1, keepdims=True)

    en = lax.broadcasted_iota(jnp.int32, (nn, 8, 2 * dim), 0)
    es = lax.broadcasted_iota(jnp.int32, (nn, 8, 2 * dim), 1)
    el = lax.broadcasted_iota(jnp.int32, (nn, 8, 2 * dim), 2)
    E3 = (en == (es * 2 + el // dim)).astype(jnp.float32)
    W13 = lax.dot_general(w1, E3, (((1,), (0,)), ((), ())),
                          preferred_element_type=jnp.float32)

    W0 = w0_r[...]
    z64 = jnp.zeros((dim, dim), jnp.float32)
    W0bd = jnp.concatenate(
        [jnp.concatenate([W0, z64], axis=1),
         jnp.concatenate([z64, W0], axis=1)], axis=0)
    b0 = b0_r[...]
    b0p = jnp.concatenate([b0, b0], axis=1)

    ev1 = ev1_r[...]
    agg1p = agg1_r[...]
    h1p = jax.nn.relu(
        lax.dot_general(ev1 + agg1p, W0bd, (((1,), (0,)), ((), ())),
                        preferred_element_type=jnp.float32) + b0p)

    def hop0_agg(xp):
        s = jnp.sum(xp.reshape(BB, 8, 2 * dim) * W13, axis=1)
        return (s[:, :dim] + s[:, dim:]) * (1.0 / nn)

    agg0 = hop0_agg(ev1)
    h0 = jax.nn.relu(
        lax.dot_general(ev0_r[...] + agg0, W0, (((1,), (0,)), ((), ())),
                        preferred_element_type=jnp.float32) + b0)

    aggf = hop0_agg(h1p)
    head = jnp.tanh(
        lax.dot_general(h0 + aggf, w1_r[...], (((1,), (0,)), ((), ())),
                        preferred_element_type=jnp.float32) + b1_r[...])

    uo = jax.nn.relu(
        lax.dot_general(u, umw_r[...], (((1,), (0,)), ((), ())),
                        preferred_element_type=jnp.float32) + umb_r[...])

    iv = iv_r[...]
    a1 = jnp.sum(head * wvv_r[...], axis=1, keepdims=True)
    a2 = jnp.sum(iv * wev_r[...], axis=1, keepdims=True)
    v_out = iv * a1 + head * a2 + bv_r[...]

    s = jnp.sum(uo * v_out, axis=1)
    sig_r[...] = (1.0 / (1.0 + jnp.exp(-s))).reshape(sig_r.shape)

    lab = lab_r[...].reshape(BB)
    bce = jnp.maximum(s, 0.0) - s * lab + jnp.log1p(jnp.exp(-jnp.abs(s)))

    @pl.when(i == 0)
    def _():
        psq = (jnp.sum(rel * rel)
               + jnp.sum(umw_r[...] ** 2) + jnp.sum(umb_r[...] ** 2)
               + jnp.sum(wvv_r[...] ** 2) + jnp.sum(wev_r[...] ** 2)
               + jnp.sum(wve_r[...] ** 2) + jnp.sum(wee_r[...] ** 2)
               + jnp.sum(bv_r[...] ** 2) + jnp.sum(be_r[...] ** 2)
               + jnp.sum(W0 * W0) + jnp.sum(b0 * b0)
               + jnp.sum(w1_r[...] ** 2) + jnp.sum(b1_r[...] ** 2))
        bce_r[...] = jnp.zeros((1, 1), jnp.float32)
        l2_r[...] = psq.reshape(1, 1)

    bce_r[...] += jnp.sum(bce).reshape(1, 1)
    l2_r[...] += (jnp.sum(uo * uo) + jnp.sum(v_out * v_out)).reshape(1, 1)


def _table_l2_kernel(a_r, b_r, c_r, acc_r):
    i = pl.program_id(0)

    @pl.when(i == 0)
    def _():
        acc_r[...] = jnp.zeros((1, 1), jnp.float32)

    a = a_r[...]
    b = b_r[...]
    c = c_r[...]
    acc_r[...] += (jnp.sum(a * a) + jnp.sum(b * b) + jnp.sum(c * c)).reshape(1, 1)


def kernel(user_indices, item_indices, labels, head_indices, adj_entity,
           adj_relation, user_emb, item_emb, entity_emb, relation_emb,
           user_mlp_W, user_mlp_b, w_vv, w_ev, w_ve, w_ee, b_v, b_e,
           agg_W0, agg_b0, agg_W1, agg_b1):
    Bn = user_indices.shape[0]
    dim = user_emb.shape[1]
    nn = adj_entity.shape[1]
    nrel = relation_emb.shape[0]

    uv, iv = _sc_stage_ui(user_indices, item_indices, user_emb, item_emb)

    e1, r1, ev0 = _sc_stage1(head_indices, adj_entity, adj_relation,
                             entity_emb)

    e1_idx = e1.reshape(Bn * nn // 128, 128)
    e2, r2, ev1 = _sc_stage2(e1_idx, adj_entity, adj_relation, entity_emb)

    PB = 512
    p_mat = pl.pallas_call(
        _p_kernel,
        grid=(Bn // PB,),
        in_specs=[
            pl.BlockSpec((PB, dim), lambda i: (i, 0)),
            pl.BlockSpec((nrel, dim), lambda i: (0, 0)),
        ],
        out_specs=pl.BlockSpec((PB, nrel), lambda i: (i, 0)),
        out_shape=jax.ShapeDtypeStruct((Bn, nrel), jnp.float32),
    )(uv, relation_emb)

    e2_idx = e2.reshape(Bn * nn * nn // 128, 128)
    r2_g = r2.reshape(Bn * nn, nn)
    agg1 = _sc_stage3(e2_idx, r2_g, p_mat.reshape(Bn * nrel), entity_emb)

    BB = 32
    grid = Bn // BB
    ev1_p = ev1.reshape(Bn * nn * dim // 128, 128)
    agg1_p = agg1.reshape(Bn * nn * dim // 128, 128)
    lab_f = labels.astype(jnp.float32).reshape(grid, 1, BB)

    row = lambda x: x.reshape(1, dim)
    full = lambda shp: pl.BlockSpec(shp, lambda i: tuple(0 for _ in shp))

    sig, bce_sum, act_sq = pl.pallas_call(
        _dense_kernel,
        grid=(grid,),
        in_specs=[
            pl.BlockSpec((BB, dim), lambda i: (i, 0)),
            pl.BlockSpec((BB, dim), lambda i: (i, 0)),
            pl.BlockSpec((BB, dim), lambda i: (i, 0)),
            pl.BlockSpec((BB * nn * dim // 128, 128), lambda i: (i, 0)),
            pl.BlockSpec((BB * nn * dim // 128, 128), lambda i: (i, 0)),
            pl.BlockSpec((BB, nn), lambda i: (i, 0)),
            pl.BlockSpec((BB, nrel), lambda i: (i, 0)),
            pl.BlockSpec((1, 1, BB), lambda i: (i, 0, 0)),
            full((nrel, dim)),
            full((dim, dim)),
            full((1, dim)),
            full((1, dim)),
            full((1, dim)),
            full((1, dim)),
            full((1, dim)),
            full((1, dim)),
            full((1, dim)),
            full((dim, dim)),
            full((1, dim)),
            full((dim, dim)),
            full((1, dim)),
        ],
        out_specs=[
            pl.BlockSpec((1, 1, BB), lambda i: (i, 0, 0)),
            pl.BlockSpec((1, 1), lambda i: (0, 0)),
            pl.BlockSpec((1, 1), lambda i: (0, 0)),
        ],
        out_shape=[
            jax.ShapeDtypeStruct((grid, 1, BB), jnp.float32),
            jax.ShapeDtypeStruct((1, 1), jnp.float32),
            jax.ShapeDtypeStruct((1, 1), jnp.float32),
        ],
    )(uv, iv, ev0, ev1_p, agg1_p, r1, p_mat, lab_f,
      relation_emb, user_mlp_W, row(user_mlp_b),
      row(w_vv.reshape(dim)), row(w_ev.reshape(dim)),
      row(w_ve.reshape(dim)), row(w_ee.reshape(dim)),
      row(b_v), row(b_e),
      agg_W0, row(agg_b0), agg_W1, row(agg_b1))

    RB = 2000
    tgrid = user_emb.shape[0] // RB
    tab_sq = pl.pallas_call(
        _table_l2_kernel,
        grid=(tgrid,),
        in_specs=[
            pl.BlockSpec((RB, dim), lambda i: (i, 0)),
            pl.BlockSpec((RB, dim), lambda i: (i, 0)),
            pl.BlockSpec((RB, dim), lambda i: (i, 0)),
        ],
        out_specs=pl.BlockSpec((1, 1), lambda i: (0, 0)),
        out_shape=jax.ShapeDtypeStruct((1, 1), jnp.float32),
    )(user_emb, item_emb, entity_emb)

    scores_normalized = sig.reshape(Bn)
    total_sq = act_sq[0, 0] + tab_sq[0, 0]
    loss = bce_sum[0, 0] / Bn + (0.5 * L2W) * total_sq
    return (scores_normalized, loss)


# speedup vs baseline: 7.3793x; 1.1000x over previous
import functools

import jax
import jax.numpy as jnp
from jax import lax
from jax.experimental import pallas as pl
from jax.experimental.pallas import tpu as pltpu
from jax.experimental.pallas import tpu_sc as plsc

NC, NS = 2, 16
NW = NC * NS
L2W = 1e-06


def _sc_mesh():
    return plsc.VectorSubcoreMesh(core_axis_name="c", subcore_axis_name="s",
                                  num_cores=NC, num_subcores=NS)


_SC_PARAMS = pltpu.CompilerParams(use_tc_tiling_on_sc=False)
_SC_PARAMS_V = pltpu.CompilerParams(use_tc_tiling_on_sc=False,
                                    needs_layout_passes=False)


def _wid():
    return lax.axis_index("s") * NC + lax.axis_index("c")


def _sc_stage1(head, adj_e, adj_r, ent_emb):
    Bn = head.shape[0]
    nn = adj_e.shape[1]
    dim = ent_emb.shape[1]
    bpw = Bn // NW

    out_type = (
        jax.ShapeDtypeStruct((Bn, nn), jnp.int32),
        jax.ShapeDtypeStruct((Bn, nn), jnp.int32),
        jax.ShapeDtypeStruct((Bn, dim), jnp.float32),
    )

    @functools.partial(
        pl.kernel, out_type=out_type, mesh=_sc_mesh(),
        compiler_params=_SC_PARAMS,
        scratch_types=[
            pltpu.VMEM((bpw,), jnp.int32),
            pltpu.VMEM((bpw, nn), jnp.int32),
            pltpu.VMEM((bpw, dim), jnp.float32),
            pltpu.SemaphoreType.DMA,
        ],
    )
    def k(head_h, adj_e_h, adj_r_h, ent_h,
          e1_h, r1_h, ev0_h, idx_v, rows_i, rows_f, sem):
        base = _wid() * bpw
        sl = pl.ds(base, bpw)
        pltpu.sync_copy(head_h.at[sl], idx_v)
        pltpu.async_copy(adj_e_h.at[idx_v], rows_i, sem).wait()
        pltpu.sync_copy(rows_i, e1_h.at[sl])
        pltpu.async_copy(adj_r_h.at[idx_v], rows_i, sem).wait()
        pltpu.sync_copy(rows_i, r1_h.at[sl])
        pltpu.async_copy(ent_h.at[idx_v], rows_f, sem).wait()
        pltpu.sync_copy(rows_f, ev0_h.at[sl])

    return k(head, adj_e, adj_r, ent_emb)


def _sc_stage_ui(uidx, iidx, usr_emb, itm_emb):
    Bn = uidx.shape[0]
    dim = usr_emb.shape[1]
    bpw = Bn // NW

    out_type = (
        jax.ShapeDtypeStruct((Bn, dim), jnp.float32),
        jax.ShapeDtypeStruct((Bn, dim), jnp.float32),
    )

    @functools.partial(
        pl.kernel, out_type=out_type, mesh=_sc_mesh(),
        compiler_params=_SC_PARAMS,
        scratch_types=[
            pltpu.VMEM((bpw,), jnp.int32),
            pltpu.VMEM((bpw, dim), jnp.float32),
            pltpu.SemaphoreType.DMA,
        ],
    )
    def k(uidx_h, iidx_h, usr_h, itm_h, uv_h, iv_h, idx_v, rows_f, sem):
        base = _wid() * bpw
        sl = pl.ds(base, bpw)
        pltpu.sync_copy(uidx_h.at[sl], idx_v)
        pltpu.async_copy(usr_h.at[idx_v], rows_f, sem).wait()
        pltpu.sync_copy(rows_f, uv_h.at[sl])
        pltpu.sync_copy(iidx_h.at[sl], idx_v)
        pltpu.async_copy(itm_h.at[idx_v], rows_f, sem).wait()
        pltpu.sync_copy(rows_f, iv_h.at[sl])

    return k(uidx, iidx, usr_emb, itm_emb)


def _sc_stage2(idx2d, adj_e, adj_r, ent_emb):
    nrows, W = idx2d.shape
    nn = adj_e.shape[1]
    dim = ent_emb.shape[1]
    rpw = nrows // NW
    nidx = nrows * W

    out_type = (
        jax.ShapeDtypeStruct((nidx, nn), jnp.int32),
        jax.ShapeDtypeStruct((nidx, nn), jnp.int32),
        jax.ShapeDtypeStruct((nidx, dim), jnp.float32),
    )

    @functools.partial(
        pl.kernel, out_type=out_type, mesh=_sc_mesh(),
        compiler_params=_SC_PARAMS,
        scratch_types=[
            pltpu.VMEM((W,), jnp.int32),
            pltpu.VMEM((W, nn), jnp.int32),
            pltpu.VMEM((W, dim), jnp.float32),
            pltpu.SemaphoreType.DMA,
        ],
    )
    def k(idx_h, adj_e_h, adj_r_h, ent_h, e2_h, r2_h, ev1_h,
          idx_v, rows_i, rows_f, sem):
        w0 = _wid() * rpw

        def body(j, _):
            row = w0 + j
            osl = pl.ds(row * W, W)
            pltpu.sync_copy(idx_h.at[row], idx_v)
            pltpu.async_copy(adj_e_h.at[idx_v], rows_i, sem).wait()
            pltpu.sync_copy(rows_i, e2_h.at[osl])
            pltpu.async_copy(adj_r_h.at[idx_v], rows_i, sem).wait()
            pltpu.sync_copy(rows_i, r2_h.at[osl])
            pltpu.async_copy(ent_h.at[idx_v], rows_f, sem).wait()
            pltpu.sync_copy(rows_f, ev1_h.at[osl])
            return 0

        lax.fori_loop(0, rpw, body, 0)

    return k(idx2d, adj_e, adj_r, ent_emb)


def _sc_stage3(idx2d, r2, p_mat, ent_emb):
    nrows, W = idx2d.shape
    ng, nn = r2.shape
    dim = ent_emb.shape[1]
    nrel = 32
    rpw = nrows // NW
    gpw = ng // NW
    gpc = W // nn
    bpw = p_mat.shape[0] // nrel // NW
    OB = 16

    out_type = jax.ShapeDtypeStruct((ng, dim), jnp.float32)

    @functools.partial(
        pl.kernel, out_type=out_type, mesh=_sc_mesh(),
        compiler_params=_SC_PARAMS_V,
        scratch_types=[
            pltpu.VMEM((gpw, nn), jnp.int32),
            pltpu.VMEM((bpw * nrel,), jnp.float32),
            pltpu.VMEM((2, W), jnp.int32),
            pltpu.VMEM((2, W, dim), jnp.float32),
            pltpu.VMEM((OB * gpc, dim), jnp.float32),
            pltpu.VMEM((W,), jnp.float32),
            pltpu.SemaphoreType.DMA,
            pltpu.SemaphoreType.DMA,
            pltpu.SemaphoreType.DMA,
        ],
    )
    def k(idx_h, r2_h, p_h, ent_h, out_h,
          r2_v, p_v, idx_v, rows_v, out_v, w_v, sem0, sem1, semo):
        wid = _wid()
        w0 = wid * rpw
        pltpu.sync_copy(r2_h.at[pl.ds(wid * gpw, gpw)], r2_v)
        pltpu.sync_copy(p_h.at[pl.ds(wid * bpw * nrel, bpw * nrel)], p_v)
        sems = (sem0, sem1)

        def start(j, slot):
            pltpu.sync_copy(idx_h.at[w0 + j], idx_v.at[slot])
            return pltpu.async_copy(ent_h.at[idx_v.at[slot]], rows_v.at[slot],
                                    sems[slot])

        start(0, 0)
        start(1, 1)

        def chunk_body(j, _):
            slot = lax.rem(j, 2)

            def do(s):
                pltpu.make_async_copy(ent_h.at[idx_v.at[s]], rows_v.at[s],
                                      sems[s]).wait()
                obase = lax.rem(j, OB) * gpc
                for g in range(gpc):
                    gl = j * gpc + g
                    bl = gl // nn
                    rvec = r2_v[gl, :]
                    svec = plsc.load_gather(
                        p_v, [bl * nrel + rvec])
                    m = jnp.max(svec, axis=0)
                    e = jnp.exp(svec - m)
                    tot = jnp.sum(e, axis=0) * float(nn)
                    w_v[pl.ds(g * nn, nn)] = e / tot
                for g in range(gpc):
                    wn = [plsc.load_gather(
                        w_v, [jnp.full((16,), g * nn + n, jnp.int32)])
                        for n in range(nn)]
                    for db in range(dim // 16):
                        sl = pl.ds(db * 16, 16)
                        t = [wn[n] * rows_v[s, g * nn + n, sl]
                             for n in range(nn)]
                        while len(t) > 1:
                            t = [t[k] + t[k + 1] for k in range(0, len(t), 2)]
                        out_v[obase + g, sl] = t[0]

                @pl.when(j + 2 < rpw)
                def _():
                    start(j + 2, s)

                @pl.when(lax.rem(j, OB) == OB - 1)
                def _():
                    pltpu.async_copy(
                        out_v,
                        out_h.at[pl.ds(wid * gpw + (j - (OB - 1)) * gpc,
                                       OB * gpc)],
                        semo).wait()

            @pl.when(slot == 0)
            def _():
                do(0)

            @pl.when(slot == 1)
            def _():
                do(1)

            return 0

        lax.fori_loop(0, rpw, chunk_body, 0)

    return k(idx2d, r2, p_mat, ent_emb)


def _p_kernel(uv_r, rel_r, p_r):
    dim = uv_r.shape[1]
    p_r[...] = lax.dot_general(uv_r[...], rel_r[...], (((1,), (1,)), ((), ())),
                               preferred_element_type=jnp.float32) * (1.0 / dim)


def _dense_kernel(uv_r, iv_r, ev0_r, ev1_r, agg1_r, r1_r, p_r, lab_r,
                  e3_r, w0bd_r, b0p_r,
                  rel_r, umw_r, umb_r, wvv_r, wev_r, wve_r, wee_r, bv_r, be_r,
                  w0_r, b0_r, w1_r, b1_r,
                  sig_r, bce_r, l2_r):
    i = pl.program_id(0)
    BB = uv_r.shape[0]
    dim = uv_r.shape[1]
    nn = 16
    nrel = rel_r.shape[0]

    u = uv_r[...]
    rel = rel_r[...]
    P = p_r[...]

    r1i = r1_r[...]
    S1 = jnp.zeros((BB, nn), jnp.float32)
    for r in range(nrel):
        S1 = jnp.where(r1i == r, P[:, r:r + 1], S1)
    m = jnp.max(S1, axis=-1, keepdims=True)
    e = jnp.exp(S1 - m)
    w1 = e / jnp.sum(e, axis=-1, keepdims=True)

    W13 = lax.dot_general(w1, e3_r[...], (((1,), (0,)), ((), ())),
                          preferred_element_type=jnp.float32)

    W0 = w0_r[...]
    W0bd = w0bd_r[...]
    b0 = b0_r[...]
    b0p = b0p_r[...]

    ev1 = ev1_r[...]
    agg1p = agg1_r[...]
    h1p = jax.nn.relu(
        lax.dot_general(ev1 + agg1p, W0bd, (((1,), (0,)), ((), ())),
                        preferred_element_type=jnp.float32) + b0p)

    def hop0_agg(xp):
        s = jnp.sum(xp.reshape(BB, 8, 2 * dim) * W13, axis=1)
        return (s[:, :dim] + s[:, dim:]) * (1.0 / nn)

    agg0 = hop0_agg(ev1)
    h0 = jax.nn.relu(
        lax.dot_general(ev0_r[...] + agg0, W0, (((1,), (0,)), ((), ())),
                        preferred_element_type=jnp.float32) + b0)

    aggf = hop0_agg(h1p)
    head = jnp.tanh(
        lax.dot_general(h0 + aggf, w1_r[...], (((1,), (0,)), ((), ())),
                        preferred_element_type=jnp.float32) + b1_r[...])

    uo = jax.nn.relu(
        lax.dot_general(u, umw_r[...], (((1,), (0,)), ((), ())),
                        preferred_element_type=jnp.float32) + umb_r[...])

    iv = iv_r[...]
    a1 = jnp.sum(head * wvv_r[...], axis=1, keepdims=True)
    a2 = jnp.sum(iv * wev_r[...], axis=1, keepdims=True)
    v_out = iv * a1 + head * a2 + bv_r[...]

    s = jnp.sum(uo * v_out, axis=1)
    sig_r[...] = (1.0 / (1.0 + jnp.exp(-s))).reshape(sig_r.shape)

    lab = lab_r[...].reshape(BB)
    bce = jnp.maximum(s, 0.0) - s * lab + jnp.log1p(jnp.exp(-jnp.abs(s)))

    @pl.when(i == 0)
    def _():
        psq = (jnp.sum(rel * rel)
               + jnp.sum(umw_r[...] ** 2) + jnp.sum(umb_r[...] ** 2)
               + jnp.sum(wvv_r[...] ** 2) + jnp.sum(wev_r[...] ** 2)
               + jnp.sum(wve_r[...] ** 2) + jnp.sum(wee_r[...] ** 2)
               + jnp.sum(bv_r[...] ** 2) + jnp.sum(be_r[...] ** 2)
               + jnp.sum(W0 * W0) + jnp.sum(b0 * b0)
               + jnp.sum(w1_r[...] ** 2) + jnp.sum(b1_r[...] ** 2))
        bce_r[...] = jnp.zeros((1, 1), jnp.float32)
        l2_r[...] = psq.reshape(1, 1)

    bce_r[...] += jnp.sum(bce).reshape(1, 1)
    l2_r[...] += (jnp.sum(uo * uo) + jnp.sum(v_out * v_out)).reshape(1, 1)


def _table_l2_kernel(a_r, b_r, c_r, acc_r):
    i = pl.program_id(0)

    @pl.when(i == 0)
    def _():
        acc_r[...] = jnp.zeros((1, 1), jnp.float32)

    a = a_r[...]
    b = b_r[...]
    c = c_r[...]
    acc_r[...] += (jnp.sum(a * a) + jnp.sum(b * b) + jnp.sum(c * c)).reshape(1, 1)


def kernel(user_indices, item_indices, labels, head_indices, adj_entity,
           adj_relation, user_emb, item_emb, entity_emb, relation_emb,
           user_mlp_W, user_mlp_b, w_vv, w_ev, w_ve, w_ee, b_v, b_e,
           agg_W0, agg_b0, agg_W1, agg_b1):
    Bn = user_indices.shape[0]
    dim = user_emb.shape[1]
    nn = adj_entity.shape[1]
    nrel = relation_emb.shape[0]

    e1, r1, ev0 = _sc_stage1(head_indices, adj_entity, adj_relation,
                             entity_emb)

    e1_idx = e1.reshape(Bn * nn // 128, 128)
    e2, r2, ev1 = _sc_stage2(e1_idx, adj_entity, adj_relation, entity_emb)

    uv, iv = _sc_stage_ui(user_indices, item_indices, user_emb, item_emb)

    PB = 512
    p_mat = pl.pallas_call(
        _p_kernel,
        grid=(Bn // PB,),
        in_specs=[
            pl.BlockSpec((PB, dim), lambda i: (i, 0)),
            pl.BlockSpec((nrel, dim), lambda i: (0, 0)),
        ],
        out_specs=pl.BlockSpec((PB, nrel), lambda i: (i, 0)),
        out_shape=jax.ShapeDtypeStruct((Bn, nrel), jnp.float32),
    )(uv, relation_emb)

    e2_idx = e2.reshape(Bn * nn * nn // 128, 128)
    r2_g = r2.reshape(Bn * nn, nn)
    agg1 = _sc_stage3(e2_idx, r2_g, p_mat.reshape(Bn * nrel), entity_emb)

    BB = 64
    grid = Bn // BB
    en = jnp.arange(nn)[:, None, None]
    esl = jnp.arange(8)[None, :, None] * 2 + jnp.arange(2 * dim)[None, None, :] // dim
    e3_c = (en == esl).astype(jnp.float32)
    z64 = jnp.zeros((dim, dim), jnp.float32)
    w0bd_c = jnp.block([[agg_W0, z64], [z64, agg_W0]])
    b0p_c = jnp.concatenate([agg_b0, agg_b0]).reshape(1, 2 * dim)
    ev1_p = ev1.reshape(Bn * nn * dim // 128, 128)
    agg1_p = agg1.reshape(Bn * nn * dim // 128, 128)
    lab_f = labels.astype(jnp.float32).reshape(grid, 1, BB)

    row = lambda x: x.reshape(1, dim)
    full = lambda shp: pl.BlockSpec(shp, lambda i: tuple(0 for _ in shp))

    sig, bce_sum, act_sq = pl.pallas_call(
        _dense_kernel,
        grid=(grid,),
        in_specs=[
            pl.BlockSpec((BB, dim), lambda i: (i, 0)),
            pl.BlockSpec((BB, dim), lambda i: (i, 0)),
            pl.BlockSpec((BB, dim), lambda i: (i, 0)),
            pl.BlockSpec((BB * nn * dim // 128, 128), lambda i: (i, 0)),
            pl.BlockSpec((BB * nn * dim // 128, 128), lambda i: (i, 0)),
            pl.BlockSpec((BB, nn), lambda i: (i, 0)),
            pl.BlockSpec((BB, nrel), lambda i: (i, 0)),
            pl.BlockSpec((1, 1, BB), lambda i: (i, 0, 0)),
            full((nn, 8, 2 * dim)),
            full((2 * dim, 2 * dim)),
            full((1, 2 * dim)),
            full((nrel, dim)),
            full((dim, dim)),
            full((1, dim)),
            full((1, dim)),
            full((1, dim)),
            full((1, dim)),
            full((1, dim)),
            full((1, dim)),
            full((1, dim)),
            full((dim, dim)),
            full((1, dim)),
            full((dim, dim)),
            full((1, dim)),
        ],
        out_specs=[
            pl.BlockSpec((1, 1, BB), lambda i: (i, 0, 0)),
            pl.BlockSpec((1, 1), lambda i: (0, 0)),
            pl.BlockSpec((1, 1), lambda i: (0, 0)),
        ],
        out_shape=[
            jax.ShapeDtypeStruct((grid, 1, BB), jnp.float32),
            jax.ShapeDtypeStruct((1, 1), jnp.float32),
            jax.ShapeDtypeStruct((1, 1), jnp.float32),
        ],
    )(uv, iv, ev0, ev1_p, agg1_p, r1, p_mat, lab_f,
      e3_c, w0bd_c, b0p_c,
      relation_emb, user_mlp_W, row(user_mlp_b),
      row(w_vv.reshape(dim)), row(w_ev.reshape(dim)),
      row(w_ve.reshape(dim)), row(w_ee.reshape(dim)),
      row(b_v), row(b_e),
      agg_W0, row(agg_b0), agg_W1, row(agg_b1))

    RB = 2000
    tgrid = user_emb.shape[0] // RB
    tab_sq = pl.pallas_call(
        _table_l2_kernel,
        grid=(tgrid,),
        in_specs=[
            pl.BlockSpec((RB, dim), lambda i: (i, 0)),
            pl.BlockSpec((RB, dim), lambda i: (i, 0)),
            pl.BlockSpec((RB, dim), lambda i: (i, 0)),
        ],
        out_specs=pl.BlockSpec((1, 1), lambda i: (0, 0)),
        out_shape=jax.ShapeDtypeStruct((1, 1), jnp.float32),
    )(user_emb, item_emb, entity_emb)

    scores_normalized = sig.reshape(Bn)
    total_sq = act_sq[0, 0] + tab_sq[0, 0]
    loss = bce_sum[0, 0] / Bn + (0.5 * L2W) * total_sq
    return (scores_normalized, loss)
